# baseline (speedup 1.0000x reference)
import functools

import numpy as np
import jax
import jax.numpy as jnp
from jax.experimental import pallas as pl
from jax.experimental.pallas import tpu as pltpu
from jax.experimental.pallas import tpu_sc as plsc

_BN = float(1.0 / np.sqrt(1.0 + 1e-5))
_NPOINTS = [512, 128, 64, 16]
_RADII = [[0.05, 0.1], [0.1, 0.2], [0.2, 0.4], [0.4, 0.8]]
_NSAMPLES = [[16, 32], [16, 32], [16, 32], [16, 32]]

_f32 = jnp.float32


def _full_spec(shape):
    nd = len(shape)
    return pl.BlockSpec(shape, lambda *args: (0,) * nd)


def _fps_body(npoint, x_ref, o_ref):
    X0, X1, X2 = x_ref[0], x_ref[1], x_ref[2]
    B, N = X0.shape
    iota = jax.lax.broadcasted_iota(jnp.int32, (B, N), 1)
    iota_s = jax.lax.broadcasted_iota(jnp.int32, (B, npoint), 1)

    def body(i, carry):
        dist, far, c0a, c1a, c2a = carry
        sel = iota == far
        c0 = jnp.sum(jnp.where(sel, X0, 0.0), axis=1, keepdims=True)
        c1 = jnp.sum(jnp.where(sel, X1, 0.0), axis=1, keepdims=True)
        c2 = jnp.sum(jnp.where(sel, X2, 0.0), axis=1, keepdims=True)
        here = iota_s == i
        c0a = jnp.where(here, c0, c0a)
        c1a = jnp.where(here, c1, c1a)
        c2a = jnp.where(here, c2, c2a)
        d0 = X0 - c0
        d1 = X1 - c1
        d2 = X2 - c2
        d = d0 * d0 + d1 * d1 + d2 * d2
        dist = jnp.minimum(dist, d)
        mx = jnp.max(dist, axis=1, keepdims=True)
        far = jnp.min(jnp.where(dist == mx, iota, N), axis=1, keepdims=True)
        return dist, far, c0a, c1a, c2a

    z = jnp.zeros((B, npoint), _f32)
    init = (jnp.full((B, N), 1e10, _f32), jnp.zeros((B, 1), jnp.int32), z, z, z)
    _, _, c0a, c1a, c2a = jax.lax.fori_loop(0, npoint, body, init)
    o_ref[0] = c0a
    o_ref[1] = c1a
    o_ref[2] = c2a


def _fps(xyzT, npoint):
    _, B, N = xyzT.shape
    return pl.pallas_call(
        functools.partial(_fps_body, npoint),
        out_shape=jax.ShapeDtypeStruct((3, B, npoint), _f32),
    )(xyzT)


def _ballq_body(N, St, ns1, ns2, r1s, r2s, nxy_ref, xT_ref, i1_ref, i2_ref):
    s = nxy_ref[0]
    s0, s1, s2 = s[:, 0:1], s[:, 1:2], s[:, 2:3]
    d = xT_ref[0]
    d0, d1, d2 = d[0:1, :], d[1:2, :], d[2:3, :]
    ss = s0 * s0 + s1 * s1 + s2 * s2
    dd = d0 * d0 + d1 * d1 + d2 * d2
    dot = s0 * d0 + s1 * d1 + s2 * d2
    sqd = ss + dd - 2.0 * dot
    iota = jax.lax.broadcasted_iota(jnp.int32, (St, N), 1).astype(_f32)
    fN = float(N)
    for ns, rs, ref in ((ns1, r1s, i1_ref), (ns2, r2s, i2_ref)):
        cand = jnp.where(sqd <= rs, iota, fN)
        first = jnp.min(cand, axis=1, keepdims=True)
        kio = jax.lax.broadcasted_iota(jnp.int32, (St, ns), 1)
        acc0 = jnp.where(kio == 0, first, jnp.zeros((St, ns), _f32))
        cand1 = jnp.where(cand == first, fN, cand)

        def step(k, c):
            cand_c, acc_c = c
            m = jnp.min(cand_c, axis=1, keepdims=True)
            v = jnp.where(m >= fN, first, m)
            acc_c = jnp.where(kio == k, v, acc_c)
            cand_c = jnp.where(cand_c == m, fN, cand_c)
            return cand_c, acc_c

        _, acc = jax.lax.fori_loop(1, ns, step, (cand1, acc0))
        ref[0] = acc.astype(jnp.int32)


def _ballq(new_xyz, xyzT, r1, r2, ns1, ns2, St):
    B, S, _ = new_xyz.shape
    N = xyzT.shape[2]
    grid = (B, S // St)
    return pl.pallas_call(
        functools.partial(_ballq_body, N, St, ns1, ns2, r1 * r1, r2 * r2),
        grid=grid,
        in_specs=[
            pl.BlockSpec((1, St, 3), lambda b, s: (b, s, 0)),
            pl.BlockSpec((1, 3, N), lambda b, s: (b, 0, 0)),
        ],
        out_specs=[
            pl.BlockSpec((1, St, ns1), lambda b, s: (b, s, 0)),
            pl.BlockSpec((1, St, ns2), lambda b, s: (b, s, 0)),
        ],
        out_shape=[
            jax.ShapeDtypeStruct((B, S, ns1), jnp.int32),
            jax.ShapeDtypeStruct((B, S, ns2), jnp.int32),
        ],
        compiler_params=pltpu.CompilerParams(
            dimension_semantics=("parallel", "parallel")),
    )(new_xyz, xyzT)


def _sc_gather(table, idx):
    R = idx.shape[0]
    D = table.shape[1]
    info = plsc.get_sparse_core_info()
    nc, nsub = info.num_cores, info.num_subcores
    nw = nc * nsub
    b_per_w = R // nw
    ck = b_per_w
    while ck * D * 4 > 196608:
        ck //= 2
    nchunks = b_per_w // ck
    mesh = plsc.VectorSubcoreMesh(core_axis_name="c", subcore_axis_name="s")

    @functools.partial(
        pl.kernel, mesh=mesh,
        out_type=jax.ShapeDtypeStruct((R, D), _f32),
        scratch_types=[
            pltpu.VMEM((ck,), jnp.int32),
            pltpu.VMEM((ck, D), _f32),
            pltpu.SemaphoreType.DMA,
        ],
    )
    def k(table_hbm, idx_hbm, out_hbm, idx_v, rows_v, sem):
        wid = jax.lax.axis_index("s") * nc + jax.lax.axis_index("c")
        base = wid * b_per_w
        for c in range(nchunks):
            off = base + c * ck
            pltpu.sync_copy(idx_hbm.at[pl.ds(off, ck)], idx_v)
            pltpu.async_copy(table_hbm.at[idx_v], rows_v, sem).wait()
            pltpu.sync_copy(rows_v, out_hbm.at[pl.ds(off, ck)])

    return k(table, idx)


def _sa_mlp_body(S, ns, C, H1, H2, H3, g_ref, nx_ref, w1p_ref, w1x_ref,
                 b1_ref, w2_ref, b2_ref, w3_ref, b3_ref, o_ref):
    g = g_ref[0]
    gp = g[:, :C]
    gx = g[:, C:C + 3]
    nx = nx_ref[0]
    rel = (gx.reshape(S, ns, 3) - nx[:, None, :]).reshape(S * ns, 3)
    a = (jnp.dot(gp, w1p_ref[...], preferred_element_type=_f32)
         + jnp.dot(rel, w1x_ref[...], preferred_element_type=_f32)
         + b1_ref[...])
    h1 = jnp.maximum(a * _BN, 0.0)
    h2 = jnp.maximum(
        (jnp.dot(h1, w2_ref[...], preferred_element_type=_f32) + b2_ref[...]) * _BN, 0.0)
    h3 = jnp.maximum(
        (jnp.dot(h2, w3_ref[...], preferred_element_type=_f32) + b3_ref[...]) * _BN, 0.0)
    o_ref[0] = jnp.max(h3.reshape(S, ns, H3), axis=1)


def _sa_mlp(g, C, new_xyz, layers):
    B, S, _ = new_xyz.shape
    ns = g.shape[1] // S
    D = g.shape[2]
    w1, w2, w3 = layers[0]['W'], layers[1]['W'], layers[2]['W']
    H1, H2, H3 = w1.shape[1], w2.shape[1], w3.shape[1]
    w1p, w1x = w1[:C], w1[C:]
    b1 = layers[0]['b'].reshape(1, H1)
    b2 = layers[1]['b'].reshape(1, H2)
    b3 = layers[2]['b'].reshape(1, H3)
    return pl.pallas_call(
        functools.partial(_sa_mlp_body, S, ns, C, H1, H2, H3),
        grid=(B,),
        in_specs=[
            pl.BlockSpec((1, S * ns, D), lambda b: (b, 0, 0)),
            pl.BlockSpec((1, S, 3), lambda b: (b, 0, 0)),
            _full_spec(w1p.shape), _full_spec(w1x.shape), _full_spec(b1.shape),
            _full_spec(w2.shape), _full_spec(b2.shape),
            _full_spec(w3.shape), _full_spec(b3.shape),
        ],
        out_specs=pl.BlockSpec((1, S, H3), lambda b: (b, 0, 0)),
        out_shape=jax.ShapeDtypeStruct((B, S, H3), _f32),
        compiler_params=pltpu.CompilerParams(
            dimension_semantics=("parallel",)),
    )(g, new_xyz, w1p, w1x, b1, w2, b2, w3, b3)


def _fp_interp(x1, x2T, p2_ref, T, n2):
    s0, s1, s2 = x1[:, 0:1], x1[:, 1:2], x1[:, 2:3]
    d0, d1, d2 = x2T[0:1, :], x2T[1:2, :], x2T[2:3, :]
    sqd = ((s0 * s0 + s1 * s1 + s2 * s2)
           + (d0 * d0 + d1 * d1 + d2 * d2)
           - 2.0 * (s0 * d0 + s1 * d1 + s2 * d2))
    iota = jax.lax.broadcasted_iota(jnp.int32, (T, n2), 1)
    cur = sqd
    ds, ams = [], []
    for _ in range(3):
        m = jnp.min(cur, axis=1, keepdims=True)
        am = jnp.min(jnp.where(cur == m, iota, n2), axis=1, keepdims=True)
        ds.append(jnp.maximum(m, 0.0))
        ams.append(am)
        cur = jnp.where(iota == am, jnp.inf, cur)
    r = [1.0 / (dd + 1e-8) for dd in ds]
    tot = r[0] + r[1] + r[2]
    wm = jnp.zeros((T, n2), _f32)
    for rr, am in zip(r, ams):
        wm = jnp.where(iota == am, rr / tot, wm)
    return jnp.dot(wm, p2_ref[0], preferred_element_type=_f32)


def _fp_body_p1(T, n2, x1_ref, x2T_ref, p2_ref, p1_ref, w1a_ref, w1b_ref,
                b1_ref, w2_ref, b2_ref, o_ref):
    interp = _fp_interp(x1_ref[0], x2T_ref[0], p2_ref, T, n2)
    h = (jnp.dot(p1_ref[0], w1a_ref[...], preferred_element_type=_f32)
         + jnp.dot(interp, w1b_ref[...], preferred_element_type=_f32)
         + b1_ref[...])
    h1 = jnp.maximum(h * _BN, 0.0)
    h2 = jnp.maximum(
        (jnp.dot(h1, w2_ref[...], preferred_element_type=_f32) + b2_ref[...]) * _BN, 0.0)
    o_ref[0] = h2


def _fp_body_nop1(T, n2, x1_ref, x2T_ref, p2_ref, w1b_ref,
                  b1_ref, w2_ref, b2_ref, o_ref):
    interp = _fp_interp(x1_ref[0], x2T_ref[0], p2_ref, T, n2)
    h = jnp.dot(interp, w1b_ref[...], preferred_element_type=_f32) + b1_ref[...]
    h1 = jnp.maximum(h * _BN, 0.0)
    h2 = jnp.maximum(
        (jnp.dot(h1, w2_ref[...], preferred_element_type=_f32) + b2_ref[...]) * _BN, 0.0)
    o_ref[0] = h2


def _fp(xyz1, xyz2T, points1, points2, layers, T):
    B, n1, _ = xyz1.shape
    n2 = xyz2T.shape[2]
    C2 = points2.shape[2]
    w1, w2 = layers[0]['W'], layers[1]['W']
    H1, H2 = w1.shape[1], w2.shape[1]
    b1 = layers[0]['b'].reshape(1, H1)
    b2 = layers[1]['b'].reshape(1, H2)
    grid = (B, n1 // T)
    common_in = [
        pl.BlockSpec((1, T, 3), lambda b, t: (b, t, 0)),
        pl.BlockSpec((1, 3, n2), lambda b, t: (b, 0, 0)),
        pl.BlockSpec((1, n2, C2), lambda b, t: (b, 0, 0)),
    ]
    out_spec = pl.BlockSpec((1, T, H2), lambda b, t: (b, t, 0))
    out_shape = jax.ShapeDtypeStruct((B, n1, H2), _f32)
    if points1 is None:
        w1b = w1
        return pl.pallas_call(
            functools.partial(_fp_body_nop1, T, n2),
            grid=grid,
            in_specs=common_in + [_full_spec(w1b.shape), _full_spec(b1.shape),
                                  _full_spec(w2.shape), _full_spec(b2.shape)],
            out_specs=out_spec,
            out_shape=out_shape,
            compiler_params=pltpu.CompilerParams(
                dimension_semantics=("parallel", "parallel")),
        )(xyz1, xyz2T, points2, w1b, b1, w2, b2)
    C1 = points1.shape[2]
    w1a, w1b = w1[:C1], w1[C1:]
    return pl.pallas_call(
        functools.partial(_fp_body_p1, T, n2),
        grid=grid,
        in_specs=common_in + [
            pl.BlockSpec((1, T, C1), lambda b, t: (b, t, 0)),
            _full_spec(w1a.shape), _full_spec(w1b.shape), _full_spec(b1.shape),
            _full_spec(w2.shape), _full_spec(b2.shape)],
        out_specs=out_spec,
        out_shape=out_shape,
        compiler_params=pltpu.CompilerParams(
            dimension_semantics=("parallel", "parallel")),
    )(xyz1, xyz2T, points2, points1, w1a, w1b, b1, w2, b2)


def _head_body(sep, w1_ref, b1_ref, w2_ref, b2_ref, pf_ref, o_ref):
    pf = pf_ref[0]
    mx = jnp.max(pf[:, :sep], axis=0, keepdims=True)
    mn = jnp.mean(pf[:, sep:], axis=0, keepdims=True)
    gf = jnp.concatenate([mx, mn], axis=1)
    h = jnp.maximum(jnp.dot(gf, w1_ref[...], preferred_element_type=_f32)
                    + b1_ref[...], 0.0)
    o_ref[0] = jnp.dot(h, w2_ref[...], preferred_element_type=_f32) + b2_ref[...]


def _head(pf, layers):
    B, N, C = pf.shape
    w1, w2 = layers[0]['W'], layers[1]['W']
    b1 = layers[0]['b'].reshape(1, -1)
    b2 = layers[1]['b'].reshape(1, -1)
    out = pl.pallas_call(
        functools.partial(_head_body, 128),
        grid=(B,),
        in_specs=[
            _full_spec(w1.shape), _full_spec(b1.shape),
            _full_spec(w2.shape), _full_spec(b2.shape),
            pl.BlockSpec((1, N, C), lambda b: (b, 0, 0)),
        ],
        out_specs=pl.BlockSpec((1, 1, b2.shape[1]), lambda b: (b, 0, 0)),
        out_shape=jax.ShapeDtypeStruct((B, 1, b2.shape[1]), _f32),
        compiler_params=pltpu.CompilerParams(
            dimension_semantics=("parallel",)),
    )(w1, b1, w2, b2, pf)
    return out.reshape(B, -1)


def _gather(points, idx):
    return jax.vmap(lambda p, i: p[i])(points, idx)


def kernel(xyz, rgb, state, params):
    B, N = xyz.shape[0], xyz.shape[1]
    mean_xyz = jnp.sum(xyz, axis=1, keepdims=True) / N
    xyz_c = xyz - mean_xyz
    feats = jnp.concatenate([
        jnp.broadcast_to(mean_xyz, (B, N, 3)), rgb, xyz_c,
        jnp.broadcast_to(state[:, None, :], (B, N, state.shape[-1]))], axis=-1)

    l_xyz = [xyz_c]
    l_points = [feats]
    cur_xyz, cur_pts = xyz_c, feats
    for i in range(4):
        S = _NPOINTS[i]
        (r1, r2), (ns1, ns2) = _RADII[i], _NSAMPLES[i]
        n = cur_xyz.shape[1]
        St = 128 if S >= 128 else S
        nxT = _fps(jnp.transpose(cur_xyz, (2, 0, 1)), S)
        new_xyz = jnp.transpose(nxT, (1, 2, 0))
        xyzT = jnp.transpose(cur_xyz, (0, 2, 1))
        idx1, idx2 = _ballq(new_xyz, xyzT, r1, r2, ns1, ns2, St)
        C = cur_pts.shape[2]
        D = ((C + 3 + 127) // 128) * 128
        table = jnp.concatenate(
            [cur_pts, cur_xyz,
             jnp.zeros((B, n, D - C - 3), _f32)], axis=-1).reshape(B * n, D)
        boff = (jnp.arange(B, dtype=jnp.int32) * n)[:, None, None]
        outs = []
        for idx, ns, layers in ((idx1, ns1, params['sa%d' % (i + 1)][0]),
                                (idx2, ns2, params['sa%d' % (i + 1)][1])):
            idxf = (idx + boff).reshape(B * S * ns)
            g = _sc_gather(table, idxf).reshape(B, S * ns, D)
            outs.append(_sa_mlp(g, C, new_xyz, layers))
        cur_xyz = new_xyz
        cur_pts = jnp.concatenate(outs, axis=-1)
        l_xyz.append(cur_xyz)
        l_points.append(cur_pts)

    for li, name in ((3, 'fp4'), (2, 'fp3'), (1, 'fp2')):
        l_points[li] = _fp(l_xyz[li], jnp.transpose(l_xyz[li + 1], (0, 2, 1)),
                           l_points[li], l_points[li + 1], params[name],
                           l_xyz[li].shape[1])
    l0 = _fp(l_xyz[0], jnp.transpose(l_xyz[1], (0, 2, 1)), None,
             l_points[1], params['fp1'], min(512, N))

    return _head(l0, params['mlp'])


# speedup vs baseline: 1.1126x; 1.1126x over previous
import functools

import numpy as np
import jax
import jax.numpy as jnp
from jax.experimental import pallas as pl
from jax.experimental.pallas import tpu as pltpu
from jax.experimental.pallas import tpu_sc as plsc

_BN = float(1.0 / np.sqrt(1.0 + 1e-5))
_NPOINTS = [512, 128, 64, 16]
_RADII = [[0.05, 0.1], [0.1, 0.2], [0.2, 0.4], [0.4, 0.8]]
_NSAMPLES = [[16, 32], [16, 32], [16, 32], [16, 32]]

_f32 = jnp.float32


def _full_spec(shape):
    nd = len(shape)
    return pl.BlockSpec(shape, lambda *args: (0,) * nd)


def _fps_body(npoint, x_ref, o_ref):
    X0, X1, X2 = x_ref[0], x_ref[1], x_ref[2]
    B, N = X0.shape
    iota = jax.lax.broadcasted_iota(jnp.int32, (B, N), 1)
    iota_s = jax.lax.broadcasted_iota(jnp.int32, (B, npoint), 1)

    def body(i, carry):
        dist, far, c0a, c1a, c2a = carry
        sel = iota == far
        c0 = jnp.sum(jnp.where(sel, X0, 0.0), axis=1, keepdims=True)
        c1 = jnp.sum(jnp.where(sel, X1, 0.0), axis=1, keepdims=True)
        c2 = jnp.sum(jnp.where(sel, X2, 0.0), axis=1, keepdims=True)
        here = iota_s == i
        c0a = jnp.where(here, c0, c0a)
        c1a = jnp.where(here, c1, c1a)
        c2a = jnp.where(here, c2, c2a)
        d0 = X0 - c0
        d1 = X1 - c1
        d2 = X2 - c2
        d = d0 * d0 + d1 * d1 + d2 * d2
        dist = jnp.minimum(dist, d)
        mx = jnp.max(dist, axis=1, keepdims=True)
        far = jnp.min(jnp.where(dist == mx, iota, N), axis=1, keepdims=True)
        return dist, far, c0a, c1a, c2a

    z = jnp.zeros((B, npoint), _f32)
    init = (jnp.full((B, N), 1e10, _f32), jnp.zeros((B, 1), jnp.int32), z, z, z)
    _, _, c0a, c1a, c2a = jax.lax.fori_loop(0, npoint, body, init)
    o_ref[0] = c0a
    o_ref[1] = c1a
    o_ref[2] = c2a


def _fps(xyzT, npoint):
    _, B, N = xyzT.shape
    return pl.pallas_call(
        functools.partial(_fps_body, npoint),
        out_shape=jax.ShapeDtypeStruct((3, B, npoint), _f32),
    )(xyzT)


def _ballq_body(N, St, ns1, ns2, r1s, r2s, nxy_ref, xT_ref, i1_ref, i2_ref):
    s = nxy_ref[0]
    s0, s1, s2 = s[:, 0:1], s[:, 1:2], s[:, 2:3]
    d = xT_ref[0]
    d0, d1, d2 = d[0:1, :], d[1:2, :], d[2:3, :]
    ss = s0 * s0 + s1 * s1 + s2 * s2
    dd = d0 * d0 + d1 * d1 + d2 * d2
    dot = s0 * d0 + s1 * d1 + s2 * d2
    sqd = ss + dd - 2.0 * dot
    def prefix_sum(x):
        s = 1
        while s < N:
            x = x + jnp.concatenate(
                [jnp.zeros((St, s), x.dtype), x[:, :N - s]], axis=1)
            s *= 2
        return x

    for ns, rs, ref in ((ns1, r1s, i1_ref), (ns2, r2s, i2_ref)):
        rank = prefix_sum((sqd <= rs).astype(jnp.int32))
        count = jnp.max(rank, axis=1, keepdims=True)
        kio = jax.lax.broadcasted_iota(jnp.int32, (St, ns), 1)

        def step(k, acc_c):
            cnt = jnp.sum((rank <= k).astype(jnp.int32), axis=1, keepdims=True)
            return jnp.where(kio == k, cnt, acc_c)

        acc = jax.lax.fori_loop(0, ns, step, jnp.zeros((St, ns), jnp.int32))
        ref[0] = jnp.where(kio < count, acc, acc[:, 0:1])


def _ballq(new_xyz, xyzT, r1, r2, ns1, ns2, St):
    B, S, _ = new_xyz.shape
    N = xyzT.shape[2]
    grid = (B, S // St)
    return pl.pallas_call(
        functools.partial(_ballq_body, N, St, ns1, ns2, r1 * r1, r2 * r2),
        grid=grid,
        in_specs=[
            pl.BlockSpec((1, St, 3), lambda b, s: (b, s, 0)),
            pl.BlockSpec((1, 3, N), lambda b, s: (b, 0, 0)),
        ],
        out_specs=[
            pl.BlockSpec((1, St, ns1), lambda b, s: (b, s, 0)),
            pl.BlockSpec((1, St, ns2), lambda b, s: (b, s, 0)),
        ],
        out_shape=[
            jax.ShapeDtypeStruct((B, S, ns1), jnp.int32),
            jax.ShapeDtypeStruct((B, S, ns2), jnp.int32),
        ],
        compiler_params=pltpu.CompilerParams(
            dimension_semantics=("parallel", "parallel")),
    )(new_xyz, xyzT)


def _sc_gather(table, idx):
    R = idx.shape[0]
    D = table.shape[1]
    info = plsc.get_sparse_core_info()
    nc, nsub = info.num_cores, info.num_subcores
    nw = nc * nsub
    b_per_w = R // nw
    ck = b_per_w
    while ck * D * 4 > 196608:
        ck //= 2
    nchunks = b_per_w // ck
    mesh = plsc.VectorSubcoreMesh(core_axis_name="c", subcore_axis_name="s")

    @functools.partial(
        pl.kernel, mesh=mesh,
        out_type=jax.ShapeDtypeStruct((R, D), _f32),
        scratch_types=[
            pltpu.VMEM((ck,), jnp.int32),
            pltpu.VMEM((ck, D), _f32),
            pltpu.SemaphoreType.DMA,
        ],
    )
    def k(table_hbm, idx_hbm, out_hbm, idx_v, rows_v, sem):
        wid = jax.lax.axis_index("s") * nc + jax.lax.axis_index("c")
        base = wid * b_per_w
        for c in range(nchunks):
            off = base + c * ck
            pltpu.sync_copy(idx_hbm.at[pl.ds(off, ck)], idx_v)
            pltpu.async_copy(table_hbm.at[idx_v], rows_v, sem).wait()
            pltpu.sync_copy(rows_v, out_hbm.at[pl.ds(off, ck)])

    return k(table, idx)


def _sa_mlp_body(S, ns, C, H1, H2, H3, g_ref, nx_ref, w1p_ref, w1x_ref,
                 b1_ref, w2_ref, b2_ref, w3_ref, b3_ref, o_ref):
    g = g_ref[0]
    gp = g[:, :C]
    gx = g[:, C:C + 3]
    nx = nx_ref[0]
    rel = (gx.reshape(S, ns, 3) - nx[:, None, :]).reshape(S * ns, 3)
    a = (jnp.dot(gp, w1p_ref[...], preferred_element_type=_f32)
         + jnp.dot(rel, w1x_ref[...], preferred_element_type=_f32)
         + b1_ref[...])
    h1 = jnp.maximum(a * _BN, 0.0)
    h2 = jnp.maximum(
        (jnp.dot(h1, w2_ref[...], preferred_element_type=_f32) + b2_ref[...]) * _BN, 0.0)
    h3 = jnp.maximum(
        (jnp.dot(h2, w3_ref[...], preferred_element_type=_f32) + b3_ref[...]) * _BN, 0.0)
    o_ref[0] = jnp.max(h3.reshape(S, ns, H3), axis=1)


def _sa_mlp(g, C, new_xyz, layers):
    B, S, _ = new_xyz.shape
    ns = g.shape[1] // S
    D = g.shape[2]
    w1, w2, w3 = layers[0]['W'], layers[1]['W'], layers[2]['W']
    H1, H2, H3 = w1.shape[1], w2.shape[1], w3.shape[1]
    w1p, w1x = w1[:C], w1[C:]
    b1 = layers[0]['b'].reshape(1, H1)
    b2 = layers[1]['b'].reshape(1, H2)
    b3 = layers[2]['b'].reshape(1, H3)
    return pl.pallas_call(
        functools.partial(_sa_mlp_body, S, ns, C, H1, H2, H3),
        grid=(B,),
        in_specs=[
            pl.BlockSpec((1, S * ns, D), lambda b: (b, 0, 0)),
            pl.BlockSpec((1, S, 3), lambda b: (b, 0, 0)),
            _full_spec(w1p.shape), _full_spec(w1x.shape), _full_spec(b1.shape),
            _full_spec(w2.shape), _full_spec(b2.shape),
            _full_spec(w3.shape), _full_spec(b3.shape),
        ],
        out_specs=pl.BlockSpec((1, S, H3), lambda b: (b, 0, 0)),
        out_shape=jax.ShapeDtypeStruct((B, S, H3), _f32),
        compiler_params=pltpu.CompilerParams(
            dimension_semantics=("parallel",)),
    )(g, new_xyz, w1p, w1x, b1, w2, b2, w3, b3)


def _fp_interp(x1, x2T, p2_ref, T, n2):
    s0, s1, s2 = x1[:, 0:1], x1[:, 1:2], x1[:, 2:3]
    d0, d1, d2 = x2T[0:1, :], x2T[1:2, :], x2T[2:3, :]
    sqd = ((s0 * s0 + s1 * s1 + s2 * s2)
           + (d0 * d0 + d1 * d1 + d2 * d2)
           - 2.0 * (s0 * d0 + s1 * d1 + s2 * d2))
    iota = jax.lax.broadcasted_iota(jnp.int32, (T, n2), 1)
    cur = sqd
    ds, ams = [], []
    for _ in range(3):
        m = jnp.min(cur, axis=1, keepdims=True)
        am = jnp.min(jnp.where(cur == m, iota, n2), axis=1, keepdims=True)
        ds.append(jnp.maximum(m, 0.0))
        ams.append(am)
        cur = jnp.where(iota == am, jnp.inf, cur)
    r = [1.0 / (dd + 1e-8) for dd in ds]
    tot = r[0] + r[1] + r[2]
    wm = jnp.zeros((T, n2), _f32)
    for rr, am in zip(r, ams):
        wm = jnp.where(iota == am, rr / tot, wm)
    return jnp.dot(wm, p2_ref[0], preferred_element_type=_f32)


def _fp_body_p1(T, n2, x1_ref, x2T_ref, p2_ref, p1_ref, w1a_ref, w1b_ref,
                b1_ref, w2_ref, b2_ref, o_ref):
    interp = _fp_interp(x1_ref[0], x2T_ref[0], p2_ref, T, n2)
    h = (jnp.dot(p1_ref[0], w1a_ref[...], preferred_element_type=_f32)
         + jnp.dot(interp, w1b_ref[...], preferred_element_type=_f32)
         + b1_ref[...])
    h1 = jnp.maximum(h * _BN, 0.0)
    h2 = jnp.maximum(
        (jnp.dot(h1, w2_ref[...], preferred_element_type=_f32) + b2_ref[...]) * _BN, 0.0)
    o_ref[0] = h2


def _fp_body_nop1(T, n2, x1_ref, x2T_ref, p2_ref, w1b_ref,
                  b1_ref, w2_ref, b2_ref, o_ref):
    interp = _fp_interp(x1_ref[0], x2T_ref[0], p2_ref, T, n2)
    h = jnp.dot(interp, w1b_ref[...], preferred_element_type=_f32) + b1_ref[...]
    h1 = jnp.maximum(h * _BN, 0.0)
    h2 = jnp.maximum(
        (jnp.dot(h1, w2_ref[...], preferred_element_type=_f32) + b2_ref[...]) * _BN, 0.0)
    o_ref[0] = h2


def _fp(xyz1, xyz2T, points1, points2, layers, T):
    B, n1, _ = xyz1.shape
    n2 = xyz2T.shape[2]
    C2 = points2.shape[2]
    w1, w2 = layers[0]['W'], layers[1]['W']
    H1, H2 = w1.shape[1], w2.shape[1]
    b1 = layers[0]['b'].reshape(1, H1)
    b2 = layers[1]['b'].reshape(1, H2)
    grid = (B, n1 // T)
    common_in = [
        pl.BlockSpec((1, T, 3), lambda b, t: (b, t, 0)),
        pl.BlockSpec((1, 3, n2), lambda b, t: (b, 0, 0)),
        pl.BlockSpec((1, n2, C2), lambda b, t: (b, 0, 0)),
    ]
    out_spec = pl.BlockSpec((1, T, H2), lambda b, t: (b, t, 0))
    out_shape = jax.ShapeDtypeStruct((B, n1, H2), _f32)
    if points1 is None:
        w1b = w1
        return pl.pallas_call(
            functools.partial(_fp_body_nop1, T, n2),
            grid=grid,
            in_specs=common_in + [_full_spec(w1b.shape), _full_spec(b1.shape),
                                  _full_spec(w2.shape), _full_spec(b2.shape)],
            out_specs=out_spec,
            out_shape=out_shape,
            compiler_params=pltpu.CompilerParams(
                dimension_semantics=("parallel", "parallel")),
        )(xyz1, xyz2T, points2, w1b, b1, w2, b2)
    C1 = points1.shape[2]
    w1a, w1b = w1[:C1], w1[C1:]
    return pl.pallas_call(
        functools.partial(_fp_body_p1, T, n2),
        grid=grid,
        in_specs=common_in + [
            pl.BlockSpec((1, T, C1), lambda b, t: (b, t, 0)),
            _full_spec(w1a.shape), _full_spec(w1b.shape), _full_spec(b1.shape),
            _full_spec(w2.shape), _full_spec(b2.shape)],
        out_specs=out_spec,
        out_shape=out_shape,
        compiler_params=pltpu.CompilerParams(
            dimension_semantics=("parallel", "parallel")),
    )(xyz1, xyz2T, points2, points1, w1a, w1b, b1, w2, b2)


def _head_body(sep, w1_ref, b1_ref, w2_ref, b2_ref, pf_ref, o_ref):
    pf = pf_ref[0]
    mx = jnp.max(pf[:, :sep], axis=0, keepdims=True)
    mn = jnp.mean(pf[:, sep:], axis=0, keepdims=True)
    gf = jnp.concatenate([mx, mn], axis=1)
    h = jnp.maximum(jnp.dot(gf, w1_ref[...], preferred_element_type=_f32)
                    + b1_ref[...], 0.0)
    o_ref[0] = jnp.dot(h, w2_ref[...], preferred_element_type=_f32) + b2_ref[...]


def _head(pf, layers):
    B, N, C = pf.shape
    w1, w2 = layers[0]['W'], layers[1]['W']
    b1 = layers[0]['b'].reshape(1, -1)
    b2 = layers[1]['b'].reshape(1, -1)
    out = pl.pallas_call(
        functools.partial(_head_body, 128),
        grid=(B,),
        in_specs=[
            _full_spec(w1.shape), _full_spec(b1.shape),
            _full_spec(w2.shape), _full_spec(b2.shape),
            pl.BlockSpec((1, N, C), lambda b: (b, 0, 0)),
        ],
        out_specs=pl.BlockSpec((1, 1, b2.shape[1]), lambda b: (b, 0, 0)),
        out_shape=jax.ShapeDtypeStruct((B, 1, b2.shape[1]), _f32),
        compiler_params=pltpu.CompilerParams(
            dimension_semantics=("parallel",)),
    )(w1, b1, w2, b2, pf)
    return out.reshape(B, -1)


def _gather(points, idx):
    return jax.vmap(lambda p, i: p[i])(points, idx)


def kernel(xyz, rgb, state, params):
    B, N = xyz.shape[0], xyz.shape[1]
    mean_xyz = jnp.sum(xyz, axis=1, keepdims=True) / N
    xyz_c = xyz - mean_xyz
    feats = jnp.concatenate([
        jnp.broadcast_to(mean_xyz, (B, N, 3)), rgb, xyz_c,
        jnp.broadcast_to(state[:, None, :], (B, N, state.shape[-1]))], axis=-1)

    l_xyz = [xyz_c]
    l_points = [feats]
    cur_xyz, cur_pts = xyz_c, feats
    for i in range(4):
        S = _NPOINTS[i]
        (r1, r2), (ns1, ns2) = _RADII[i], _NSAMPLES[i]
        n = cur_xyz.shape[1]
        St = 128 if S >= 128 else S
        nxT = _fps(jnp.transpose(cur_xyz, (2, 0, 1)), S)
        new_xyz = jnp.transpose(nxT, (1, 2, 0))
        xyzT = jnp.transpose(cur_xyz, (0, 2, 1))
        idx1, idx2 = _ballq(new_xyz, xyzT, r1, r2, ns1, ns2, St)
        C = cur_pts.shape[2]
        D = ((C + 3 + 127) // 128) * 128
        table = jnp.concatenate(
            [cur_pts, cur_xyz,
             jnp.zeros((B, n, D - C - 3), _f32)], axis=-1).reshape(B * n, D)
        boff = (jnp.arange(B, dtype=jnp.int32) * n)[:, None, None]
        outs = []
        for idx, ns, layers in ((idx1, ns1, params['sa%d' % (i + 1)][0]),
                                (idx2, ns2, params['sa%d' % (i + 1)][1])):
            idxf = jnp.minimum((idx + boff).reshape(B * S * ns), B * n - 1)
            g = _sc_gather(table, idxf).reshape(B, S * ns, D)
            outs.append(_sa_mlp(g, C, new_xyz, layers))
        cur_xyz = new_xyz
        cur_pts = jnp.concatenate(outs, axis=-1)
        l_xyz.append(cur_xyz)
        l_points.append(cur_pts)

    for li, name in ((3, 'fp4'), (2, 'fp3'), (1, 'fp2')):
        l_points[li] = _fp(l_xyz[li], jnp.transpose(l_xyz[li + 1], (0, 2, 1)),
                           l_points[li], l_points[li + 1], params[name],
                           l_xyz[li].shape[1])
    l0 = _fp(l_xyz[0], jnp.transpose(l_xyz[1], (0, 2, 1)), None,
             l_points[1], params['fp1'], min(512, N))

    return _head(l0, params['mlp'])
